# baseline (speedup 1.0000x reference)
import functools

import jax
import jax.numpy as jnp
from jax import lax
from jax.experimental import pallas as pl
from jax.experimental.pallas import tpu as pltpu
from jax.experimental.pallas import tpu_sc as plsc

BATCH = 16384
EMBED_DIM = 64
CHUNK = 128
LANES = 16


def _make_kernel():
    info = plsc.get_sparse_core_info()
    num_cores, num_subcores = info.num_cores, info.num_subcores
    nw = num_cores * num_subcores
    b_per_w = BATCH // nw
    n_chunks = b_per_w // CHUNK

    out_t = tuple(
        jax.ShapeDtypeStruct((BATCH, EMBED_DIM), jnp.float32) for _ in range(4)
    )

    scratch = (
        [pltpu.VMEM((b_per_w,), jnp.int32) for _ in range(4)]
        + [pltpu.VMEM((CHUNK, EMBED_DIM), jnp.float32)]
        + [pltpu.SemaphoreType.DMA,
           pltpu.SemaphoreType.DMA]
        + [pltpu.SemaphoreType.DMA for _ in range(LANES)]
    )

    @functools.partial(
        pl.kernel,
        mesh=plsc.VectorSubcoreMesh(core_axis_name="c", subcore_axis_name="s"),
        out_type=out_t,
        compiler_params=pltpu.CompilerParams(needs_layout_passes=False),
        scratch_types=scratch,
    )
    def k(head_h, rel_h, tail_h, ts_h, ent_t, rel_t, ts_t,
          out0, out1, out2, out3, *sc):
        idx_refs = sc[0:4]
        row_buf = sc[4]
        isem, wsem = sc[5:7]
        gsems = sc[7:7 + LANES]

        wid = lax.axis_index("s") * num_cores + lax.axis_index("c")
        row_base = wid * b_per_w

        idx_loads = [
            pltpu.async_copy(src.at[pl.ds(row_base, b_per_w)], dst, isem)
            for src, dst in zip((head_h, rel_h, tail_h, ts_h), idx_refs)
        ]
        for cp in idx_loads:
            cp.wait()

        tables = (ent_t, rel_t, ent_t, ts_t)
        outs = (out0, out1, out2, out3)

        for l in range(4):
            table, idx_ref, out = tables[l], idx_refs[l], outs[l]

            def chunk_body(c, _, table=table, idx_ref=idx_ref, out=out):
                off = c * CHUNK
                for g in range(CHUNK // LANES):
                    iv = idx_ref[pl.ds(off + g * LANES, LANES)]
                    for j in range(LANES):
                        pltpu.async_copy(
                            table.at[iv[j]],
                            row_buf.at[g * LANES + j],
                            gsems[j],
                        )
                for j in range(LANES):
                    pltpu.make_async_copy(
                        table.at[pl.ds(0, CHUNK // LANES)],
                        row_buf.at[pl.ds(0, CHUNK // LANES)],
                        gsems[j],
                    ).wait()
                pltpu.async_copy(
                    row_buf, out.at[pl.ds(row_base + off, CHUNK)], wsem
                ).wait()
                return 0

            lax.fori_loop(0, n_chunks, chunk_body, 0)

    return k


_sc_lookup = _make_kernel()


def kernel(head, relation, tail, timestamp, entity_table, relation_table, timestamp_table):
    return _sc_lookup(
        head, relation, tail, timestamp,
        entity_table, relation_table, timestamp_table,
    )


# speedup vs baseline: 1.2582x; 1.2582x over previous
import functools

import jax
import jax.numpy as jnp
from jax import lax
from jax.experimental import pallas as pl
from jax.experimental.pallas import tpu as pltpu
from jax.experimental.pallas import tpu_sc as plsc

BATCH = 16384
EMBED_DIM = 64
CHUNK = 128
LANES = 16

_info = plsc.get_sparse_core_info()
_NC, _NS = _info.num_cores, _info.num_subcores
_NW = _NC * _NS
_B_PER_W = BATCH // _NW
_N_CHUNKS = _B_PER_W // CHUNK


def _make_entity_kernel():
    out_t = tuple(
        jax.ShapeDtypeStruct((BATCH, EMBED_DIM), jnp.float32) for _ in range(2)
    )
    scratch = (
        [pltpu.VMEM((_B_PER_W,), jnp.int32) for _ in range(2)]
        + [pltpu.VMEM((CHUNK, EMBED_DIM), jnp.float32)]
        + [pltpu.SemaphoreType.DMA,
           pltpu.SemaphoreType.DMA,
           pltpu.SemaphoreType.DMA]
    )

    @functools.partial(
        pl.kernel,
        mesh=plsc.VectorSubcoreMesh(core_axis_name="c", subcore_axis_name="s"),
        out_type=out_t,
        compiler_params=pltpu.CompilerParams(needs_layout_passes=False),
        scratch_types=scratch,
    )
    def k(head_h, tail_h, ent_t, out0, out2, ih, it, row_buf, isem, gsem, wsem):
        wid = lax.axis_index("s") * _NC + lax.axis_index("c")
        row_base = wid * _B_PER_W

        for cp in [
            pltpu.async_copy(src.at[pl.ds(row_base, _B_PER_W)], dst, isem)
            for src, dst in ((head_h, ih), (tail_h, it))
        ]:
            cp.wait()

        for idx_ref, out in ((ih, out0), (it, out2)):

            def chunk_body(c, _, idx_ref=idx_ref, out=out):
                off = c * CHUNK
                for g in range(CHUNK // LANES):
                    iv = idx_ref[pl.ds(off + g * LANES, LANES)]
                    for j in range(LANES):
                        pltpu.async_copy(
                            ent_t.at[iv[j]],
                            row_buf.at[g * LANES + j],
                            gsem,
                        )
                pltpu.make_async_copy(
                    ent_t.at[pl.ds(0, CHUNK)], row_buf, gsem
                ).wait()
                pltpu.async_copy(
                    row_buf, out.at[pl.ds(row_base + off, CHUNK)], wsem
                ).wait()
                return 0

            lax.fori_loop(0, _N_CHUNKS, chunk_body, 0)

    return k


def _make_small_kernel():
    out_t = tuple(
        jax.ShapeDtypeStruct((BATCH, EMBED_DIM), jnp.float32) for _ in range(2)
    )
    scratch = (
        [pltpu.VMEM((_N_CHUNKS, CHUNK), jnp.int32) for _ in range(2)]
        + [pltpu.VMEM((CHUNK, EMBED_DIM), jnp.float32) for _ in range(2)]
        + [pltpu.SemaphoreType.DMA,
           pltpu.SemaphoreType.DMA,
           pltpu.SemaphoreType.DMA]
    )

    @functools.partial(
        pl.kernel,
        mesh=plsc.VectorSubcoreMesh(core_axis_name="c", subcore_axis_name="s"),
        out_type=out_t,
        compiler_params=pltpu.CompilerParams(use_tc_tiling_on_sc=False),
        scratch_types=scratch,
    )
    def k(rel_h, ts_h, rel_t, ts_t, out1, out3, ir, its, buf0, buf1,
          isem, gsem, wsem):
        wid = lax.axis_index("s") * _NC + lax.axis_index("c")
        row_base = wid * _B_PER_W
        chunk_base = wid * _N_CHUNKS

        for cp in [
            pltpu.async_copy(src.at[pl.ds(chunk_base, _N_CHUNKS)], dst, isem)
            for src, dst in ((rel_h, ir), (ts_h, its))
        ]:
            cp.wait()

        bufs = (buf0, buf1)
        pending = [None, None]
        tasks = [
            (idx.at[c], table, out.at[pl.ds(row_base + c * CHUNK, CHUNK)])
            for idx, table, out in ((ir, rel_t, out1), (its, ts_t, out3))
            for c in range(_N_CHUNKS)
        ]
        for t, (idx, table, dst) in enumerate(tasks):
            s = t % 2
            if pending[s] is not None:
                pending[s].wait()
            pltpu.async_copy(table.at[idx], bufs[s], gsem).wait()
            pending[s] = pltpu.async_copy(bufs[s], dst, wsem)
        for s in range(2):
            if pending[s] is not None:
                pending[s].wait()

    return k


_ent_lookup = _make_entity_kernel()
_small_lookup = _make_small_kernel()


def kernel(head, relation, tail, timestamp, entity_table, relation_table, timestamp_table):
    idx2 = lambda a: a.reshape(BATCH // CHUNK, CHUNK)
    out0, out2 = _ent_lookup(head, tail, entity_table)
    out1, out3 = _small_lookup(
        idx2(relation), idx2(timestamp), relation_table, timestamp_table
    )
    return (out0, out1, out2, out3)
